# baseline (speedup 1.0000x reference)
import functools

import jax
import jax.numpy as jnp
from jax import lax
from jax.experimental import pallas as pl
from jax.experimental.pallas import tpu as pltpu
from jax.experimental.pallas import tpu_sc as plsc

_NC = 2
_NS = 16
_NW = _NC * _NS
_CHUNK = 112


def _sc_gather(h_all, c_all, idx_r, n_pad, k, hs):
    mesh = plsc.VectorSubcoreMesh(
        core_axis_name="c", subcore_axis_name="s",
        num_cores=_NC, num_subcores=_NS)
    out_sds = jax.ShapeDtypeStruct((n_pad, hs), jnp.float32)

    @functools.partial(
        pl.kernel,
        mesh=mesh,
        out_type=[out_sds] * 4,
        scratch_types=(
            [pltpu.VMEM((k, _CHUNK), jnp.int32)] * 2
            + [pltpu.VMEM((_CHUNK, hs), jnp.float32)] * 8
            + [pltpu.SemaphoreType.DMA] * 4
        ),
    )
    def gather_kernel(h_hbm, c_hbm, idx_hbm, oh0, oh1, oc0, oc1,
                      i0_v, i1_v, a0, a1, a2, a3, b0, b1, b2, b3,
                      gsa, gsb, ssa, ssb):
        wid = lax.axis_index("s") * _NC + lax.axis_index("c")
        pltpu.sync_copy(idx_hbm.at[0, wid], i0_v)
        pltpu.sync_copy(idx_hbm.at[1, wid], i1_v)
        outs = (oh0, oh1, oc0, oc1)
        bufs_a = (a0, a1, a2, a3)
        bufs_b = (b0, b1, b2, b3)
        tabs = (h_hbm, h_hbm, c_hbm, c_hbm)
        idxs = (i0_v, i1_v, i0_v, i1_v)
        base0 = wid * (k * _CHUNK)

        def gissue(j, bufs, sem):
            for t in range(4):
                pltpu.async_copy(tabs[t].at[idxs[t].at[j]], bufs[t], sem)

        def gwait(bufs, sem):
            for t in range(4):
                pltpu.make_async_copy(
                    tabs[t].at[idxs[t].at[0]], bufs[t], sem).wait()

        def sissue(j, bufs, sem):
            for t in range(4):
                pltpu.async_copy(
                    bufs[t], outs[t].at[pl.ds(base0 + j * _CHUNK, _CHUNK)],
                    sem)

        def sdrain(bufs, sem):
            for t in range(4):
                pltpu.make_async_copy(
                    bufs[t], outs[t].at[pl.ds(0, _CHUNK)], sem).wait()

        gissue(0, bufs_a, gsa)

        def body(it, carry):
            ja = 2 * it
            jb = ja + 1
            jn = lax.min(ja + 2, k - 1)
            gwait(bufs_a, gsa)

            @pl.when(it > 0)
            def _():
                sdrain(bufs_b, ssb)

            gissue(jb, bufs_b, gsb)
            sissue(ja, bufs_a, ssa)
            gwait(bufs_b, gsb)
            sdrain(bufs_a, ssa)
            gissue(jn, bufs_a, gsa)
            sissue(jb, bufs_b, ssb)
            return carry

        lax.fori_loop(0, k // 2, body, 0)
        gwait(bufs_a, gsa)
        sdrain(bufs_b, ssb)

    return gather_kernel(h_all, c_all, idx_r)


def _tc_fused(x, h0, h1, c0, c1, w_iou, ui0, ui1, uf0, uf1, ufb, biou,
              prev_out, off_blocks, slice_blocks, n, hs, block):
    grid = (slice_blocks,)

    def body(x_ref, h0_ref, h1_ref, c0_ref, c1_ref, wiou_ref, ui0_ref,
             ui1_ref, uf0_ref, uf1_ref, ufb_ref, biou_ref, *rest):
        out_ref = rest[-1]
        h0b = h0_ref[...]
        h1b = h1_ref[...]
        fpre = (jnp.dot(h0b, uf0_ref[...], preferred_element_type=jnp.float32)
                + jnp.dot(h1b, uf1_ref[...], preferred_element_type=jnp.float32)
                + ufb_ref[...])
        f0 = jax.nn.sigmoid(fpre[:, :hs])
        f1 = jax.nn.sigmoid(fpre[:, hs:])
        cred = f0 * c0_ref[...] + f1 * c1_ref[...]
        iou = (jnp.dot(x_ref[...], wiou_ref[...],
                       preferred_element_type=jnp.float32)
               + jnp.dot(h0b, ui0_ref[...], preferred_element_type=jnp.float32)
               + jnp.dot(h1b, ui1_ref[...], preferred_element_type=jnp.float32)
               + biou_ref[...])
        i = jax.nn.sigmoid(iou[:, :hs])
        o = jax.nn.sigmoid(iou[:, hs:2 * hs])
        u = jnp.tanh(iou[:, 2 * hs:])
        c = i * u + cred
        h = o * jnp.tanh(c)
        out_ref[:, :hs] = h
        out_ref[:, hs:] = c

    row_g = lambda i: (i + off_blocks, 0)
    row_l = lambda i: (i, 0)
    full = lambda i: (0, 0)
    in_specs = [
        pl.BlockSpec((block, x.shape[1]), row_g),
        pl.BlockSpec((block, hs), row_l),
        pl.BlockSpec((block, hs), row_l),
        pl.BlockSpec((block, hs), row_l),
        pl.BlockSpec((block, hs), row_l),
        pl.BlockSpec(w_iou.shape, full),
        pl.BlockSpec(ui0.shape, full),
        pl.BlockSpec(ui1.shape, full),
        pl.BlockSpec(uf0.shape, full),
        pl.BlockSpec(uf1.shape, full),
        pl.BlockSpec(ufb.shape, full),
        pl.BlockSpec(biou.shape, full),
    ]
    args = [x, h0, h1, c0, c1, w_iou, ui0, ui1, uf0, uf1, ufb, biou]
    aliases = {}
    if prev_out is not None:
        in_specs.append(pl.BlockSpec(memory_space=pl.ANY))
        args.append(prev_out)
        aliases = {12: 0}
    return pl.pallas_call(
        body,
        grid=grid,
        in_specs=in_specs,
        out_specs=pl.BlockSpec((block, 2 * hs), row_g),
        out_shape=jax.ShapeDtypeStruct((n, 2 * hs), jnp.float32),
        input_output_aliases=aliases,
        compiler_params=pltpu.CompilerParams(
            dimension_semantics=("arbitrary",)),
    )(*args)


_BLOCK = 512
_KS = 8


def kernel(x, h_all, c_all, child_idx, W_iou, U_iou, U_f_w, U_f_b, b_iou):
    n, _ = x.shape
    hs = h_all.shape[1]

    k = pl.cdiv(n, _NW * _CHUNK)
    k += k & 1
    n_pad = _NW * k * _CHUNK

    idx32 = child_idx.astype(jnp.int32)
    idx_t = jnp.pad(idx32.T, ((0, 0), (0, n_pad - n)))

    ks_list = []
    rem = k
    while rem > 0:
        ks = min(_KS, rem)
        ks_list.append(ks)
        rem -= ks

    ui0, ui1 = U_iou[:hs], U_iou[hs:]
    uf0, uf1 = U_f_w[:hs], U_f_w[hs:]
    ufb = U_f_b.reshape(1, 2 * hs)

    out = None
    base = 0
    for ks in ks_list:
        rows = _NW * ks * _CHUNK
        idx_r = idx_t[:, base:base + rows].reshape(2, _NW, ks, _CHUNK)
        h0, h1, c0, c1 = _sc_gather(h_all, c_all, idx_r, rows, ks, hs)
        out = _tc_fused(x, h0, h1, c0, c1, W_iou, ui0, ui1, uf0, uf1,
                        ufb, b_iou, out, base // _BLOCK, rows // _BLOCK,
                        n, hs, _BLOCK)
        base += rows
    return out


# speedup vs baseline: 1.1319x; 1.1319x over previous
import functools

import jax
import jax.numpy as jnp
from jax import lax
from jax.experimental import pallas as pl
from jax.experimental.pallas import tpu as pltpu
from jax.experimental.pallas import tpu_sc as plsc

_NC = 2
_NS = 16
_NW = _NC * _NS
_CHUNK = 112


def _sc_gather(h_all, c_all, idx_r, n_pad, k, hs):
    mesh = plsc.VectorSubcoreMesh(
        core_axis_name="c", subcore_axis_name="s",
        num_cores=_NC, num_subcores=_NS)
    out_sds = jax.ShapeDtypeStruct((n_pad, hs), jnp.float32)

    @functools.partial(
        pl.kernel,
        mesh=mesh,
        out_type=[out_sds] * 4,
        scratch_types=(
            [pltpu.VMEM((k, _CHUNK), jnp.int32)] * 2
            + [pltpu.VMEM((_CHUNK, hs), jnp.float32)] * 8
            + [pltpu.SemaphoreType.DMA] * 4
        ),
    )
    def gather_kernel(h_hbm, c_hbm, idx_hbm, oh0, oh1, oc0, oc1,
                      i0_v, i1_v, a0, a1, a2, a3, b0, b1, b2, b3,
                      gsa, gsb, ssa, ssb):
        wid = lax.axis_index("s") * _NC + lax.axis_index("c")
        pltpu.sync_copy(idx_hbm.at[0, wid], i0_v)
        pltpu.sync_copy(idx_hbm.at[1, wid], i1_v)
        outs = (oh0, oh1, oc0, oc1)
        bufs_a = (a0, a1, a2, a3)
        bufs_b = (b0, b1, b2, b3)
        tabs = (h_hbm, h_hbm, c_hbm, c_hbm)
        idxs = (i0_v, i1_v, i0_v, i1_v)
        base0 = wid * (k * _CHUNK)

        def gissue(j, bufs, sem):
            for t in range(4):
                pltpu.async_copy(tabs[t].at[idxs[t].at[j]], bufs[t], sem)

        def gwait(bufs, sem):
            for t in range(4):
                pltpu.make_async_copy(
                    tabs[t].at[idxs[t].at[0]], bufs[t], sem).wait()

        def sissue(j, bufs, sem):
            for t in range(4):
                pltpu.async_copy(
                    bufs[t], outs[t].at[pl.ds(base0 + j * _CHUNK, _CHUNK)],
                    sem)

        def sdrain(bufs, sem):
            for t in range(4):
                pltpu.make_async_copy(
                    bufs[t], outs[t].at[pl.ds(0, _CHUNK)], sem).wait()

        gissue(0, bufs_a, gsa)

        def body(it, carry):
            ja = 2 * it
            jb = ja + 1
            jn = lax.min(ja + 2, k - 1)
            gwait(bufs_a, gsa)

            @pl.when(it > 0)
            def _():
                sdrain(bufs_b, ssb)

            gissue(jb, bufs_b, gsb)
            sissue(ja, bufs_a, ssa)
            gwait(bufs_b, gsb)
            sdrain(bufs_a, ssa)
            gissue(jn, bufs_a, gsa)
            sissue(jb, bufs_b, ssb)
            return carry

        lax.fori_loop(0, k // 2, body, 0)
        gwait(bufs_a, gsa)
        sdrain(bufs_b, ssb)

    return gather_kernel(h_all, c_all, idx_r)


def _tc_fused(x, h0, h1, c0, c1, w_iou, ui0, ui1, uf0, uf1, ufb, biou,
              prev_out, off_blocks, slice_blocks, n, hs, block):
    grid = (slice_blocks,)

    def body(x_ref, h0_ref, h1_ref, c0_ref, c1_ref, wiou_ref, ui0_ref,
             ui1_ref, uf0_ref, uf1_ref, ufb_ref, biou_ref, *rest):
        out_ref = rest[-1]
        h0b = h0_ref[...]
        h1b = h1_ref[...]
        fpre = (jnp.dot(h0b, uf0_ref[...], preferred_element_type=jnp.float32)
                + jnp.dot(h1b, uf1_ref[...], preferred_element_type=jnp.float32)
                + ufb_ref[...])
        f0 = jax.nn.sigmoid(fpre[:, :hs])
        f1 = jax.nn.sigmoid(fpre[:, hs:])
        cred = f0 * c0_ref[...] + f1 * c1_ref[...]
        iou = (jnp.dot(x_ref[...], wiou_ref[...],
                       preferred_element_type=jnp.float32)
               + jnp.dot(h0b, ui0_ref[...], preferred_element_type=jnp.float32)
               + jnp.dot(h1b, ui1_ref[...], preferred_element_type=jnp.float32)
               + biou_ref[...])
        i = jax.nn.sigmoid(iou[:, :hs])
        o = jax.nn.sigmoid(iou[:, hs:2 * hs])
        u = jnp.tanh(iou[:, 2 * hs:])
        c = i * u + cred
        h = o * jnp.tanh(c)
        out_ref[:, :hs] = h
        out_ref[:, hs:] = c

    row_g = lambda i: (i + off_blocks, 0)
    row_l = lambda i: (i, 0)
    full = lambda i: (0, 0)
    in_specs = [
        pl.BlockSpec((block, x.shape[1]), row_g),
        pl.BlockSpec((block, hs), row_l),
        pl.BlockSpec((block, hs), row_l),
        pl.BlockSpec((block, hs), row_l),
        pl.BlockSpec((block, hs), row_l),
        pl.BlockSpec(w_iou.shape, full),
        pl.BlockSpec(ui0.shape, full),
        pl.BlockSpec(ui1.shape, full),
        pl.BlockSpec(uf0.shape, full),
        pl.BlockSpec(uf1.shape, full),
        pl.BlockSpec(ufb.shape, full),
        pl.BlockSpec(biou.shape, full),
    ]
    args = [x, h0, h1, c0, c1, w_iou, ui0, ui1, uf0, uf1, ufb, biou]
    aliases = {}
    if prev_out is not None:
        in_specs.append(pl.BlockSpec(memory_space=pl.ANY))
        args.append(prev_out)
        aliases = {12: 0}
    return pl.pallas_call(
        body,
        grid=grid,
        in_specs=in_specs,
        out_specs=pl.BlockSpec((block, 2 * hs), row_g),
        out_shape=jax.ShapeDtypeStruct((n, 2 * hs), jnp.float32),
        input_output_aliases=aliases,
        compiler_params=pltpu.CompilerParams(
            dimension_semantics=("arbitrary",)),
    )(*args)


_BLOCK = 1024
_KS = 8


def kernel(x, h_all, c_all, child_idx, W_iou, U_iou, U_f_w, U_f_b, b_iou):
    n, _ = x.shape
    hs = h_all.shape[1]

    k = pl.cdiv(n, _NW * _CHUNK)
    k += k & 1
    n_pad = _NW * k * _CHUNK

    idx32 = child_idx.astype(jnp.int32)
    idx_t = jnp.pad(idx32.T, ((0, 0), (0, n_pad - n)))

    ks_list = []
    rem = k
    while rem > 0:
        ks = min(_KS, rem)
        ks_list.append(ks)
        rem -= ks

    ui0, ui1 = U_iou[:hs], U_iou[hs:]
    uf0, uf1 = U_f_w[:hs], U_f_w[hs:]
    ufb = U_f_b.reshape(1, 2 * hs)

    out = None
    base = 0
    for ks in ks_list:
        rows = _NW * ks * _CHUNK
        idx_r = idx_t[:, base:base + rows].reshape(2, _NW, ks, _CHUNK)
        h0, h1, c0, c1 = _sc_gather(h_all, c_all, idx_r, rows, ks, hs)
        out = _tc_fused(x, h0, h1, c0, c1, W_iou, ui0, ui1, uf0, uf1,
                        ufb, b_iou, out, base // _BLOCK, rows // _BLOCK,
                        n, hs, _BLOCK)
        base += rows
    return out


# speedup vs baseline: 1.1544x; 1.0199x over previous
import functools

import jax
import jax.numpy as jnp
from jax import lax
from jax.experimental import pallas as pl
from jax.experimental.pallas import tpu as pltpu
from jax.experimental.pallas import tpu_sc as plsc

_NC = 2
_NS = 16
_NW = _NC * _NS
_CHUNK = 112


def _sc_gather(h_all, c_all, idx_r, n_pad, k, hs):
    mesh = plsc.VectorSubcoreMesh(
        core_axis_name="c", subcore_axis_name="s",
        num_cores=_NC, num_subcores=_NS)
    out_sds = jax.ShapeDtypeStruct((n_pad, hs), jnp.float32)

    @functools.partial(
        pl.kernel,
        mesh=mesh,
        out_type=[out_sds] * 4,
        scratch_types=(
            [pltpu.VMEM((k, _CHUNK), jnp.int32)] * 2
            + [pltpu.VMEM((_CHUNK, hs), jnp.float32)] * 8
            + [pltpu.SemaphoreType.DMA] * 4
        ),
    )
    def gather_kernel(h_hbm, c_hbm, idx_hbm, oh0, oh1, oc0, oc1,
                      i0_v, i1_v, a0, a1, a2, a3, b0, b1, b2, b3,
                      gsa, gsb, ssa, ssb):
        wid = lax.axis_index("s") * _NC + lax.axis_index("c")
        pltpu.sync_copy(idx_hbm.at[0, wid], i0_v)
        pltpu.sync_copy(idx_hbm.at[1, wid], i1_v)
        outs = (oh0, oh1, oc0, oc1)
        bufs_a = (a0, a1, a2, a3)
        bufs_b = (b0, b1, b2, b3)
        tabs = (h_hbm, h_hbm, c_hbm, c_hbm)
        idxs = (i0_v, i1_v, i0_v, i1_v)
        base0 = wid * (k * _CHUNK)

        def gissue(j, bufs, sem):
            for t in range(4):
                pltpu.async_copy(tabs[t].at[idxs[t].at[j]], bufs[t], sem)

        def gwait(bufs, sem):
            for t in range(4):
                pltpu.make_async_copy(
                    tabs[t].at[idxs[t].at[0]], bufs[t], sem).wait()

        def sissue(j, bufs, sem):
            for t in range(4):
                pltpu.async_copy(
                    bufs[t], outs[t].at[pl.ds(base0 + j * _CHUNK, _CHUNK)],
                    sem)

        def sdrain(bufs, sem):
            for t in range(4):
                pltpu.make_async_copy(
                    bufs[t], outs[t].at[pl.ds(0, _CHUNK)], sem).wait()

        gissue(0, bufs_a, gsa)

        def body(it, carry):
            ja = 2 * it
            jb = ja + 1
            jn = lax.min(ja + 2, k - 1)
            gwait(bufs_a, gsa)

            @pl.when(it > 0)
            def _():
                sdrain(bufs_b, ssb)

            gissue(jb, bufs_b, gsb)
            sissue(ja, bufs_a, ssa)
            gwait(bufs_b, gsb)
            sdrain(bufs_a, ssa)
            gissue(jn, bufs_a, gsa)
            sissue(jb, bufs_b, ssb)
            return carry

        lax.fori_loop(0, k // 2, body, 0)
        gwait(bufs_a, gsa)
        sdrain(bufs_b, ssb)

    return gather_kernel(h_all, c_all, idx_r)


def _tc_fused(x, h0, h1, c0, c1, w_iou, ui0, ui1, uf0, uf1, ufb, biou,
              prev_out, off_blocks, slice_blocks, n, hs, block):
    grid = (slice_blocks,)

    def body(x_ref, h0_ref, h1_ref, c0_ref, c1_ref, wiou_ref, ui0_ref,
             ui1_ref, uf0_ref, uf1_ref, ufb_ref, biou_ref, *rest):
        out_ref = rest[-1]
        h0b = h0_ref[...]
        h1b = h1_ref[...]
        fpre = (jnp.dot(h0b, uf0_ref[...], preferred_element_type=jnp.float32)
                + jnp.dot(h1b, uf1_ref[...], preferred_element_type=jnp.float32)
                + ufb_ref[...])
        f0 = jax.nn.sigmoid(fpre[:, :hs])
        f1 = jax.nn.sigmoid(fpre[:, hs:])
        cred = f0 * c0_ref[...] + f1 * c1_ref[...]
        iou = (jnp.dot(x_ref[...], wiou_ref[...],
                       preferred_element_type=jnp.float32)
               + jnp.dot(h0b, ui0_ref[...], preferred_element_type=jnp.float32)
               + jnp.dot(h1b, ui1_ref[...], preferred_element_type=jnp.float32)
               + biou_ref[...])
        i = jax.nn.sigmoid(iou[:, :hs])
        o = jax.nn.sigmoid(iou[:, hs:2 * hs])
        u = jnp.tanh(iou[:, 2 * hs:])
        c = i * u + cred
        h = o * jnp.tanh(c)
        out_ref[:, :hs] = h
        out_ref[:, hs:] = c

    row_g = lambda i: (i + off_blocks, 0)
    row_l = lambda i: (i, 0)
    full = lambda i: (0, 0)
    in_specs = [
        pl.BlockSpec((block, x.shape[1]), row_g),
        pl.BlockSpec((block, hs), row_l),
        pl.BlockSpec((block, hs), row_l),
        pl.BlockSpec((block, hs), row_l),
        pl.BlockSpec((block, hs), row_l),
        pl.BlockSpec(w_iou.shape, full),
        pl.BlockSpec(ui0.shape, full),
        pl.BlockSpec(ui1.shape, full),
        pl.BlockSpec(uf0.shape, full),
        pl.BlockSpec(uf1.shape, full),
        pl.BlockSpec(ufb.shape, full),
        pl.BlockSpec(biou.shape, full),
    ]
    args = [x, h0, h1, c0, c1, w_iou, ui0, ui1, uf0, uf1, ufb, biou]
    aliases = {}
    if prev_out is not None:
        in_specs.append(pl.BlockSpec(memory_space=pl.ANY))
        args.append(prev_out)
        aliases = {12: 0}
    return pl.pallas_call(
        body,
        grid=grid,
        in_specs=in_specs,
        out_specs=pl.BlockSpec((block, 2 * hs), row_g),
        out_shape=jax.ShapeDtypeStruct((n, 2 * hs), jnp.float32),
        input_output_aliases=aliases,
        compiler_params=pltpu.CompilerParams(
            dimension_semantics=("arbitrary",)),
    )(*args)


_BLOCK = 2048
_KS = 8


def kernel(x, h_all, c_all, child_idx, W_iou, U_iou, U_f_w, U_f_b, b_iou):
    n, _ = x.shape
    hs = h_all.shape[1]

    k = pl.cdiv(n, _NW * _CHUNK)
    k += k & 1
    n_pad = _NW * k * _CHUNK

    idx32 = child_idx.astype(jnp.int32)
    idx_t = jnp.pad(idx32.T, ((0, 0), (0, n_pad - n)))

    ks_list = []
    rem = k
    while rem > 0:
        ks = min(_KS, rem)
        ks_list.append(ks)
        rem -= ks

    ui0, ui1 = U_iou[:hs], U_iou[hs:]
    uf0, uf1 = U_f_w[:hs], U_f_w[hs:]
    ufb = U_f_b.reshape(1, 2 * hs)

    out = None
    base = 0
    for ks in ks_list:
        rows = _NW * ks * _CHUNK
        idx_r = idx_t[:, base:base + rows].reshape(2, _NW, ks, _CHUNK)
        h0, h1, c0, c1 = _sc_gather(h_all, c_all, idx_r, rows, ks, hs)
        out = _tc_fused(x, h0, h1, c0, c1, W_iou, ui0, ui1, uf0, uf1,
                        ufb, b_iou, out, base // _BLOCK, rows // _BLOCK,
                        n, hs, _BLOCK)
        base += rows
    return out
